# baseline (speedup 1.0000x reference)
import functools

import jax
import jax.numpy as jnp
from jax import lax
from jax.experimental import pallas as pl
from jax.experimental.pallas import tpu as pltpu
from jax.experimental.pallas import tpu_sc as plsc

_NUM_USERS = 500000
_B = 16384
_EMB = 64

_NW = 32
_ROWS_PER_W = _B // _NW
_CHUNK = 128
_NCHUNK = _ROWS_PER_W // _CHUNK
_NBUF = 6
_DEPTH = 3


def _body(mu_uv, sigma_uv, mu_g, sigma_g, users_i, items_i, negs_i,
          o_u_mu, o_u_sigma, o_v_mu, o_v_sigma, o_u_mu_g, o_u_sigma_g,
          o_n_mu, o_n_sigma,
          idx_u, idx_i, idx_n, bufs, gsems, wsems):
    nc = plsc.get_sparse_core_info().num_cores
    wid = lax.axis_index("s") * nc + lax.axis_index("c")
    base = wid * _ROWS_PER_W

    pltpu.sync_copy(users_i.at[wid], idx_u)
    pltpu.sync_copy(items_i.at[wid], idx_i)
    pltpu.sync_copy(negs_i.at[wid], idx_n)

    tasks = [
        (mu_uv, idx_u, o_u_mu),
        (sigma_uv, idx_u, o_u_sigma),
        (mu_uv, idx_i, o_v_mu),
        (sigma_uv, idx_i, o_v_sigma),
        (mu_g, idx_u, o_u_mu_g),
        (sigma_g, idx_u, o_u_sigma_g),
        (mu_uv, idx_n, o_n_mu),
        (sigma_uv, idx_n, o_n_sigma),
    ]
    jobs = [(tbl, idx, c, out) for (tbl, idx, out) in tasks
            for c in range(_NCHUNK)]

    n = len(jobs)
    g_h = [None] * n
    w_h = [None] * n
    for j in range(n + _DEPTH):
        if j < n:
            tbl, idx, c, out = jobs[j]
            b = j % _NBUF
            if j >= _NBUF:
                w_h[j - _NBUF].wait()
            g_h[j] = pltpu.make_async_copy(tbl.at[idx.at[c]],
                                           bufs.at[b], gsems.at[b])
            g_h[j].start()
        i = j - _DEPTH
        if i >= 0:
            tbl, idx, c, out = jobs[i]
            b = i % _NBUF
            g_h[i].wait()
            w_h[i] = pltpu.make_async_copy(
                bufs.at[b], out.at[pl.ds(base + c * _CHUNK, _CHUNK)],
                wsems.at[b])
            w_h[i].start()
    for i in range(n - _NBUF, n):
        w_h[i].wait()


@jax.jit
def _run(mu_uv, sigma_uv, mu_g, sigma_g, users_i, items_i, negs_i):
    out = jax.ShapeDtypeStruct((_B, _EMB), jnp.float32)
    kfn = pl.kernel(
        _body,
        out_type=(out,) * 8,
        mesh=plsc.VectorSubcoreMesh(core_axis_name="c", subcore_axis_name="s"),
        scratch_types=[
            pltpu.VMEM((_NCHUNK, _CHUNK), jnp.int32),
            pltpu.VMEM((_NCHUNK, _CHUNK), jnp.int32),
            pltpu.VMEM((_NCHUNK, _CHUNK), jnp.int32),
            pltpu.VMEM((_NBUF, _CHUNK, _EMB), jnp.float32),
            pltpu.SemaphoreType.DMA((_NBUF,)),
            pltpu.SemaphoreType.DMA((_NBUF,)),
        ],
        compiler_params=pltpu.CompilerParams(use_tc_tiling_on_sc=False),
    )
    return kfn(mu_uv, sigma_uv, mu_g, sigma_g, users_i, items_i, negs_i)


def kernel(users, items, neg_items, U_mu_g, U_sigma_g, U_and_V_mu,
           U_and_V_sigma):
    users_i = users.astype(jnp.int32).reshape(_NW, _NCHUNK, _CHUNK)
    items_i = (items.astype(jnp.int32) + _NUM_USERS).reshape(
        _NW, _NCHUNK, _CHUNK)
    negs_i = (neg_items.astype(jnp.int32) + _NUM_USERS).reshape(
        _NW, _NCHUNK, _CHUNK)
    return _run(U_and_V_mu, U_and_V_sigma, U_mu_g, U_sigma_g,
                users_i, items_i, negs_i)
